# baseline (speedup 1.0000x reference)
import jax
import jax.numpy as jnp
from jax.experimental import pallas as pl
from jax.experimental.pallas import tpu as pltpu

B, L, D = 16, 4096, 256
H, O = 512, 128
CHUNK = 1024
NC = L // CHUNK
TMAX = B * NC


def _rowsum8(x):
    n = x.shape[0]
    while n > 8:
        n //= 2
        x = x[:n] + x[n:]
    return x


def _fused_kernel(row1_ref, ch1_ref, row2_ref, ch2_ref, t12_ref,
                  l1_ref, l2_ref,
                  x1_hbm, x2_hbm, len1f_ref, len2f_ref,
                  w1_ref, b1_ref, w2_ref, b2_ref,
                  out_ref, acc1_ref, acc2_ref, e1_ref, e2_ref, t_ref):
    t1 = t12_ref[0]
    t2 = t12_ref[1]
    tmax = jnp.maximum(t1, t2)

    acc1_ref[...] = jnp.zeros_like(acc1_ref)
    acc2_ref[...] = jnp.zeros_like(acc2_ref)
    t_ref[0] = 0

    iota = jax.lax.broadcasted_iota(jnp.int32, (CHUNK, 1), 0)

    def x1_map(t):
        tc = jnp.minimum(t, t1 - 1)
        return (row1_ref[tc], ch1_ref[tc], 0)

    def x2_map(t):
        tc = jnp.minimum(t, t2 - 1)
        return (row2_ref[tc], ch2_ref[tc], 0)

    def body(x1_blk, x2_blk):
        t = t_ref[0]

        def accum(tlim, row_ref, ch_ref, len_ref, x_blk, acc_ref, e_ref):
            tc = jnp.minimum(t, tlim - 1)
            row = row_ref[tc]
            ch = ch_ref[tc]
            length = len_ref[row]
            lim = length - ch * CHUNK

            @pl.when((t < tlim) & (lim >= CHUNK))
            def _():
                acc_ref[...] += _rowsum8(x_blk[0])

            @pl.when((t < tlim) & (lim < CHUNK))
            def _():
                xm = jnp.where(iota < lim, x_blk[0], 0.0)
                acc_ref[...] += _rowsum8(xm)

            @pl.when((t < tlim) & (lim <= CHUNK))
            def _():
                e_ref[pl.ds(row, 1), :] = jnp.sum(acc_ref[...], axis=0,
                                                  keepdims=True)
                acc_ref[...] = jnp.zeros_like(acc_ref)

        accum(t1, row1_ref, ch1_ref, l1_ref, x1_blk, acc1_ref, e1_ref)
        accum(t2, row2_ref, ch2_ref, l2_ref, x2_blk, acc2_ref, e2_ref)
        t_ref[0] = t + 1

    pipeline = pltpu.emit_pipeline(
        body,
        grid=(tmax,),
        in_specs=[
            pl.BlockSpec((1, CHUNK, D), x1_map),
            pl.BlockSpec((1, CHUNK, D), x2_map),
        ],
    )
    pipeline(x1_hbm, x2_hbm)

    e1 = e1_ref[...] / len1f_ref[...]
    e2 = e2_ref[...] / len2f_ref[...]
    cat = jnp.concatenate([e1, e2, jnp.abs(e1 - e2), e1 * e2], axis=1)
    h = jnp.dot(cat, w1_ref[...], preferred_element_type=jnp.float32)
    h = jnp.maximum(h + b1_ref[...], 0.0)
    out_ref[...] = (
        jnp.dot(h, w2_ref[...], preferred_element_type=jnp.float32)
        + b2_ref[...]
    )


def _schedule(lengths):
    nc = (lengths + CHUNK - 1) // CHUNK
    t_total = jnp.sum(nc)
    starts = jnp.cumsum(nc) - nc
    row = jnp.repeat(jnp.arange(B, dtype=jnp.int32), nc,
                     total_repeat_length=TMAX)
    ch = jnp.arange(TMAX, dtype=jnp.int32) - starts[row]
    return row, ch, t_total


def kernel(X1, x1_lengths, X2, x2_lengths, W1, b1, W2, b2):
    len1f = x1_lengths.astype(jnp.float32).reshape(B, 1)
    len2f = x2_lengths.astype(jnp.float32).reshape(B, 1)
    row1, ch1, t1 = _schedule(x1_lengths)
    row2, ch2, t2 = _schedule(x2_lengths)
    t12 = jnp.stack([t1, t2]).astype(jnp.int32)

    grid_spec = pltpu.PrefetchScalarGridSpec(
        num_scalar_prefetch=7,
        grid=(1,),
        in_specs=[
            pl.BlockSpec(memory_space=pl.ANY),
            pl.BlockSpec(memory_space=pl.ANY),
            pl.BlockSpec((B, 1), lambda g, *_: (0, 0)),
            pl.BlockSpec((B, 1), lambda g, *_: (0, 0)),
            pl.BlockSpec((4 * D, H), lambda g, *_: (0, 0)),
            pl.BlockSpec((1, H), lambda g, *_: (0, 0)),
            pl.BlockSpec((H, O), lambda g, *_: (0, 0)),
            pl.BlockSpec((1, O), lambda g, *_: (0, 0)),
        ],
        out_specs=pl.BlockSpec((B, O), lambda g, *_: (0, 0)),
        scratch_shapes=[
            pltpu.VMEM((8, D), jnp.float32),
            pltpu.VMEM((8, D), jnp.float32),
            pltpu.VMEM((B, D), jnp.float32),
            pltpu.VMEM((B, D), jnp.float32),
            pltpu.SMEM((1,), jnp.int32),
        ],
    )

    return pl.pallas_call(
        _fused_kernel,
        grid_spec=grid_spec,
        out_shape=jax.ShapeDtypeStruct((B, O), jnp.float32),
        compiler_params=pltpu.CompilerParams(
            dimension_semantics=("arbitrary",),
        ),
    )(row1, ch1, row2, ch2, t12, x1_lengths, x2_lengths,
      X1, X2, len1f, len2f, W1, b1.reshape(1, H), W2, b2.reshape(1, O))


# speedup vs baseline: 1.3378x; 1.3378x over previous
import jax
import jax.numpy as jnp
from jax.experimental import pallas as pl
from jax.experimental.pallas import tpu as pltpu

B, L, D = 16, 4096, 256
H, O = 512, 128
CHUNK = 1024
NC = L // CHUNK


def _num_chunks(length):
    return (length + CHUNK - 1) // CHUNK


def _rowsum8(x):
    n = x.shape[0]
    while n > 8:
        n //= 2
        x = x[:n] + x[n:]
    return x


def _fused_kernel(l1_ref, l2_ref,
                  x1_ref, x2_ref, len1f_ref, len2f_ref,
                  w1_ref, b1_ref, w2_ref, b2_ref,
                  out_ref, acc1_ref, acc2_ref, e1_ref, e2_ref):
    i = pl.program_id(0)
    j = pl.program_id(1)
    base = j * CHUNK

    def accum(len_ref, x_ref, acc_ref, e_ref):
        length = len_ref[i]

        @pl.when(j == 0)
        def _():
            acc_ref[...] = jnp.zeros_like(acc_ref)

        @pl.when(base + CHUNK <= length)
        def _():
            acc_ref[...] += _rowsum8(x_ref[0])

        @pl.when((base < length) & (length < base + CHUNK))
        def _():
            row = jax.lax.broadcasted_iota(jnp.int32, (CHUNK, 1), 0) + base
            xm = jnp.where(row < length, x_ref[0], 0.0)
            acc_ref[...] += _rowsum8(xm)

        @pl.when(j == NC - 1)
        def _():
            e_ref[pl.ds(i, 1), :] = jnp.sum(acc_ref[...], axis=0,
                                            keepdims=True)

    accum(l1_ref, x1_ref, acc1_ref, e1_ref)
    accum(l2_ref, x2_ref, acc2_ref, e2_ref)

    @pl.when((i == B - 1) & (j == NC - 1))
    def _():
        e1 = e1_ref[...] / len1f_ref[...]
        e2 = e2_ref[...] / len2f_ref[...]
        cat = jnp.concatenate([e1, e2, jnp.abs(e1 - e2), e1 * e2], axis=1)
        h = jnp.dot(cat, w1_ref[...], preferred_element_type=jnp.float32)
        h = jnp.maximum(h + b1_ref[...], 0.0)
        out_ref[...] = (
            jnp.dot(h, w2_ref[...], preferred_element_type=jnp.float32)
            + b2_ref[...]
        )


def kernel(X1, x1_lengths, X2, x2_lengths, W1, b1, W2, b2):
    len1f = x1_lengths.astype(jnp.float32).reshape(B, 1)
    len2f = x2_lengths.astype(jnp.float32).reshape(B, 1)

    def x_spec(which):
        def index_map(i, j, l1, l2):
            lens = l1 if which == 0 else l2
            return (i, jnp.minimum(j, _num_chunks(lens[i]) - 1), 0)
        return pl.BlockSpec((1, CHUNK, D), index_map)

    const = lambda shape: pl.BlockSpec(shape, lambda i, j, l1, l2: (0,) * len(shape))

    grid_spec = pltpu.PrefetchScalarGridSpec(
        num_scalar_prefetch=2,
        grid=(B, NC),
        in_specs=[
            x_spec(0),
            x_spec(1),
            const((B, 1)),
            const((B, 1)),
            const((4 * D, H)),
            const((1, H)),
            const((H, O)),
            const((1, O)),
        ],
        out_specs=const((B, O)),
        scratch_shapes=[
            pltpu.VMEM((8, D), jnp.float32),
            pltpu.VMEM((8, D), jnp.float32),
            pltpu.VMEM((B, D), jnp.float32),
            pltpu.VMEM((B, D), jnp.float32),
        ],
    )

    return pl.pallas_call(
        _fused_kernel,
        grid_spec=grid_spec,
        out_shape=jax.ShapeDtypeStruct((B, O), jnp.float32),
        compiler_params=pltpu.CompilerParams(
            dimension_semantics=("arbitrary", "arbitrary"),
        ),
    )(x1_lengths, x2_lengths, X1, X2, len1f, len2f,
      W1, b1.reshape(1, H), W2, b2.reshape(1, O))


# speedup vs baseline: 1.3777x; 1.0298x over previous
import jax
import jax.numpy as jnp
from jax.experimental import pallas as pl
from jax.experimental.pallas import tpu as pltpu

B, L, D = 16, 4096, 256
H, O = 512, 128
CHUNK = 1024
NC = L // CHUNK


def _num_chunks(length):
    return (length + CHUNK - 1) // CHUNK


NACC = 8


def _chunksum(load):
    m = CHUNK // NACC
    parts = []
    for a in range(NACC):
        s = load(a * m)
        for k in range(1, m // 8):
            s = s + load(a * m + k * 8)
        parts.append(s)
    while len(parts) > 1:
        parts = [parts[p] + parts[p + 1] for p in range(0, len(parts), 2)]
    return parts[0]


def _fused_kernel(l1_ref, l2_ref,
                  x1_ref, x2_ref, len1f_ref, len2f_ref,
                  w1_ref, b1_ref, w2_ref, b2_ref,
                  out_ref, acc1_ref, acc2_ref, e1_ref, e2_ref):
    i = pl.program_id(0)
    j = pl.program_id(1)
    base = j * CHUNK

    def accum(len_ref, x_ref, acc_ref, e_ref):
        length = len_ref[i]

        @pl.when(j == 0)
        def _():
            acc_ref[...] = jnp.zeros_like(acc_ref)

        @pl.when(base + CHUNK <= length)
        def _():
            acc_ref[...] += _chunksum(
                lambda lo: x_ref[0, pl.ds(lo, 8), :])

        @pl.when((base < length) & (length < base + CHUNK))
        def _():
            lim = length - base
            iota8 = jax.lax.broadcasted_iota(jnp.int32, (8, 1), 0)

            def load_masked(lo):
                return jnp.where(iota8 + lo < lim,
                                 x_ref[0, pl.ds(lo, 8), :], 0.0)

            acc_ref[...] += _chunksum(load_masked)

        @pl.when(j == NC - 1)
        def _():
            e_ref[pl.ds(i, 1), :] = jnp.sum(acc_ref[...], axis=0,
                                            keepdims=True)

    accum(l1_ref, x1_ref, acc1_ref, e1_ref)
    accum(l2_ref, x2_ref, acc2_ref, e2_ref)

    @pl.when((i == B - 1) & (j == NC - 1))
    def _():
        e1 = e1_ref[...] / len1f_ref[...]
        e2 = e2_ref[...] / len2f_ref[...]
        cat = jnp.concatenate([e1, e2, jnp.abs(e1 - e2), e1 * e2], axis=1)
        h = jnp.dot(cat, w1_ref[...], preferred_element_type=jnp.float32)
        h = jnp.maximum(h + b1_ref[...], 0.0)
        out_ref[...] = (
            jnp.dot(h, w2_ref[...], preferred_element_type=jnp.float32)
            + b2_ref[...]
        )


def kernel(X1, x1_lengths, X2, x2_lengths, W1, b1, W2, b2):
    len1f = x1_lengths.astype(jnp.float32).reshape(B, 1)
    len2f = x2_lengths.astype(jnp.float32).reshape(B, 1)

    def x_spec(which):
        def index_map(i, j, l1, l2):
            lens = l1 if which == 0 else l2
            return (i, jnp.minimum(j, _num_chunks(lens[i]) - 1), 0)
        return pl.BlockSpec((1, CHUNK, D), index_map)

    const = lambda shape: pl.BlockSpec(shape, lambda i, j, l1, l2: (0,) * len(shape))

    grid_spec = pltpu.PrefetchScalarGridSpec(
        num_scalar_prefetch=2,
        grid=(B, NC),
        in_specs=[
            x_spec(0),
            x_spec(1),
            const((B, 1)),
            const((B, 1)),
            const((4 * D, H)),
            const((1, H)),
            const((H, O)),
            const((1, O)),
        ],
        out_specs=const((B, O)),
        scratch_shapes=[
            pltpu.VMEM((8, D), jnp.float32),
            pltpu.VMEM((8, D), jnp.float32),
            pltpu.VMEM((B, D), jnp.float32),
            pltpu.VMEM((B, D), jnp.float32),
        ],
    )

    return pl.pallas_call(
        _fused_kernel,
        grid_spec=grid_spec,
        out_shape=jax.ShapeDtypeStruct((B, O), jnp.float32),
        compiler_params=pltpu.CompilerParams(
            dimension_semantics=("arbitrary", "arbitrary"),
        ),
    )(x1_lengths, x2_lengths, X1, X2, len1f, len2f,
      W1, b1.reshape(1, H), W2, b2.reshape(1, O))
